# baseline (speedup 1.0000x reference)
import jax
import jax.numpy as jnp
from jax import lax
from jax.experimental import pallas as pl
from jax.experimental.pallas import tpu as pltpu

N, D, T = 100000, 32, 8
NC = T + 1
B = 8192
NB = -(-N // B)
NP = NB * B
RB = B // 128
R = NP // 128
CH = NB
VCH = N // B
NEG = -1e30
BIGI = 2**31 - 1


def _tree_max(v):
    while v.shape[0] > 8:
        h = v.shape[0] // 2
        v = jnp.maximum(v[:h, :], v[h:, :])
    return jnp.max(v, axis=0).reshape(1, 128)


def _body(tgt_sm, xt_blk, u_ref, xt_any, out_ref, *rest):
    scrs = rest[0:NC]
    ps = rest[NC:2 * NC]
    wt, tiles, sem = rest[2 * NC:]
    k = pl.program_id(0)

    @pl.when(k == 0)
    def _init():
        copies = []
        bases = []
        for i in range(T):
            base = pl.multiple_of((tgt_sm[i] // 128) * 128, 128)
            bases.append(base)
            c = pltpu.make_async_copy(
                xt_any.at[:, pl.ds(base, 128)],
                tiles.at[i],
                sem,
            )
            c.start()
            copies.append(c)
        for c in copies:
            c.wait()
        wt[...] = jnp.zeros((D, 16), jnp.float32)
        wt[:, 0:1] = jnp.transpose(u_ref[...])
        lane2 = lax.broadcasted_iota(jnp.int32, (D, 128), 1)
        for i in range(T):
            tl = tgt_sm[i] - bases[i]
            col = jnp.sum(jnp.where(lane2 == tl, tiles[i], 0.0), axis=1)
            wt[:, 1 + i:2 + i] = col.reshape(D, 1)

    x = xt_blk[...]
    s = lax.dot_general(wt[...], x, (((0,), (0,)), ((), ())),
                        precision=lax.Precision.HIGHEST,
                        preferred_element_type=jnp.float32)
    s3 = s.reshape(16, RB, 128)
    for c in range(NC):
        sc = s3[c]
        scrs[c][pl.ds(k * RB, RB), :] = sc
        ps[c][pl.ds(k, 1), :] = _tree_max(sc)

    @pl.when(k == NB - 1)
    def _select():
        lane1 = lax.broadcasted_iota(jnp.int32, (1, 128), 1)
        chunk_iota = lax.broadcasted_iota(jnp.int32, (CH, 128), 0)
        rowi = lax.broadcasted_iota(jnp.int32, (RB, 128), 0)
        gloc = rowi * 128 + lax.broadcasted_iota(jnp.int32, (RB, 128), 1)

        gtail = VCH * B + gloc
        for c in range(NC):
            tailv = scrs[c][pl.ds(VCH * RB, RB), :]
            ps[c][pl.ds(VCH, 1), :] = _tree_max(
                jnp.where(gtail < N, tailv, NEG))

        def refresh_chunk(scr, p, ci):
            sch = scr[pl.ds(ci * RB, RB), :]
            p[pl.ds(ci, 1), :] = _tree_max(
                jnp.where(ci * B + gloc < N, sch, NEG))

        def pick(scr, p):
            pm = p[...]
            m = jnp.max(pm)
            ci = jnp.min(jnp.where(pm == m, chunk_iota, BIGI))
            sch = scr[pl.ds(ci * RB, RB), :]
            hit = (sch == m) & (ci * B + gloc < N)
            g = ci * B + jnp.min(jnp.where(hit, gloc, BIGI))
            r = g // 128
            rowv = scr[pl.ds(r, 1), :]
            scr[pl.ds(r, 1), :] = jnp.where(lane1 == g % 128, NEG, rowv)
            gl = g - ci * B
            v = jnp.where((gloc != gl) & (ci * B + gloc < N), sch, NEG)
            p[pl.ds(ci, 1), :] = _tree_max(v)
            return g, m

        def exclude(scr, p, g, cond=None):
            r = g // 128
            hit = lane1 == g % 128
            if cond is not None:
                hit = hit & cond
            rowv = scr[pl.ds(r, 1), :]
            scr[pl.ds(r, 1), :] = jnp.where(hit, NEG, rowv)
            refresh_chunk(scr, p, r // RB)

        def score_at(g):
            rowv = scrs[0][pl.ds(g // 128, 1), :]
            return jnp.sum(jnp.where(lane1 == g % 128, rowv, 0.0))

        tops = []
        for _ in range(6):
            tops.append(pick(scrs[0], ps[0]))
        for g, m in tops:
            rowv = scrs[0][pl.ds(g // 128, 1), :]
            scrs[0][pl.ds(g // 128, 1), :] = jnp.where(lane1 == g % 128, m,
                                                       rowv)

        loss = jnp.float32(0.0)
        for t in range(T):
            tt = tgt_sm[t]
            s_t = score_at(tt)

            in5 = tops[0][0] == tt
            for i in range(1, 5):
                in5 = in5 | (tops[i][0] == tt)
            contrib = jnp.float32(0.0)
            for i in range(5):
                contrib += jnp.where(tops[i][0] == tt, 0.0,
                                     jax.nn.sigmoid(tops[i][1] - s_t))
            contrib += jnp.where(in5, jax.nn.sigmoid(tops[5][1] - s_t), 0.0)

            scr, p = scrs[1 + t], ps[1 + t]
            exclude(scr, p, tt)
            for i in range(5):
                exclude(scr, p, tops[i][0])
            exclude(scr, p, tops[5][0], cond=in5)
            for _ in range(5):
                g, _m = pick(scr, p)
                contrib += jax.nn.sigmoid(score_at(g) - s_t)

            loss += contrib
        out_ref[...] = jnp.broadcast_to(loss, (1, 1))


def kernel(items_emb, user_emb, target_items):
    xt = jnp.transpose(items_emb)
    grid_spec = pltpu.PrefetchScalarGridSpec(
        num_scalar_prefetch=1,
        grid=(NB,),
        in_specs=[
            pl.BlockSpec((D, B), lambda k, tgt: (0, k)),
            pl.BlockSpec((1, D), lambda k, tgt: (0, 0)),
            pl.BlockSpec(memory_space=pltpu.MemorySpace.HBM),
        ],
        out_specs=pl.BlockSpec((1, 1), lambda k, tgt: (0, 0)),
        scratch_shapes=(
            [pltpu.VMEM((R, 128), jnp.float32) for _ in range(NC)]
            + [pltpu.VMEM((CH, 128), jnp.float32) for _ in range(NC)]
            + [pltpu.VMEM((D, 16), jnp.float32),
               pltpu.VMEM((T, D, 128), jnp.float32),
               pltpu.SemaphoreType.DMA]
        ),
    )
    out = pl.pallas_call(
        _body,
        grid_spec=grid_spec,
        out_shape=jax.ShapeDtypeStruct((1, 1), jnp.float32),
    )(target_items, xt, user_emb, xt)
    return out[0, 0]


# speedup vs baseline: 1.0416x; 1.0416x over previous
import jax
import jax.numpy as jnp
from jax import lax
from jax.experimental import pallas as pl
from jax.experimental.pallas import tpu as pltpu

N, D, T = 100000, 32, 8
NC = T + 1
B = 16384
NB = -(-N // B)
NP = NB * B
RB = B // 128
R = NP // 128
CH = NB
VCH = N // B
NEG = -1e30
BIGI = 2**31 - 1


def _tree_max(v):
    while v.shape[0] > 8:
        h = v.shape[0] // 2
        v = jnp.maximum(v[:h, :], v[h:, :])
    return jnp.max(v, axis=0).reshape(1, 128)


def _body(tgt_sm, xt_blk, u_ref, xt_any, out_ref, *rest):
    scrs = rest[0:NC]
    ps = rest[NC:2 * NC]
    wt, tiles, sem = rest[2 * NC:]
    k = pl.program_id(0)

    @pl.when(k == 0)
    def _init():
        copies = []
        bases = []
        for i in range(T):
            base = pl.multiple_of((tgt_sm[i] // 128) * 128, 128)
            bases.append(base)
            c = pltpu.make_async_copy(
                xt_any.at[:, pl.ds(base, 128)],
                tiles.at[i],
                sem,
            )
            c.start()
            copies.append(c)
        for c in copies:
            c.wait()
        wt[...] = jnp.zeros((D, 16), jnp.float32)
        wt[:, 0:1] = jnp.transpose(u_ref[...])
        lane2 = lax.broadcasted_iota(jnp.int32, (D, 128), 1)
        for i in range(T):
            tl = tgt_sm[i] - bases[i]
            col = jnp.sum(jnp.where(lane2 == tl, tiles[i], 0.0), axis=1)
            wt[:, 1 + i:2 + i] = col.reshape(D, 1)

    x = xt_blk[...]
    s = lax.dot_general(wt[...], x, (((0,), (0,)), ((), ())),
                        precision=lax.Precision.HIGHEST,
                        preferred_element_type=jnp.float32)
    s3 = s.reshape(16, RB, 128)
    for c in range(NC):
        sc = s3[c]
        scrs[c][pl.ds(k * RB, RB), :] = sc
        ps[c][pl.ds(k, 1), :] = _tree_max(sc)

    @pl.when(k == NB - 1)
    def _select():
        lane1 = lax.broadcasted_iota(jnp.int32, (1, 128), 1)
        chunk_iota = lax.broadcasted_iota(jnp.int32, (CH, 128), 0)
        rowi = lax.broadcasted_iota(jnp.int32, (RB, 128), 0)
        gloc = rowi * 128 + lax.broadcasted_iota(jnp.int32, (RB, 128), 1)

        gtail = VCH * B + gloc
        for c in range(NC):
            tailv = scrs[c][pl.ds(VCH * RB, RB), :]
            ps[c][pl.ds(VCH, 1), :] = _tree_max(
                jnp.where(gtail < N, tailv, NEG))

        def refresh_chunk(scr, p, ci):
            sch = scr[pl.ds(ci * RB, RB), :]
            p[pl.ds(ci, 1), :] = _tree_max(
                jnp.where(ci * B + gloc < N, sch, NEG))

        def pick(scr, p):
            pm = p[...]
            m = jnp.max(pm)
            ci = jnp.min(jnp.where(pm == m, chunk_iota, BIGI))
            sch = scr[pl.ds(ci * RB, RB), :]
            hit = (sch == m) & (ci * B + gloc < N)
            g = ci * B + jnp.min(jnp.where(hit, gloc, BIGI))
            r = g // 128
            rowv = scr[pl.ds(r, 1), :]
            scr[pl.ds(r, 1), :] = jnp.where(lane1 == g % 128, NEG, rowv)
            gl = g - ci * B
            v = jnp.where((gloc != gl) & (ci * B + gloc < N), sch, NEG)
            p[pl.ds(ci, 1), :] = _tree_max(v)
            return g, m

        def exclude(scr, p, g, cond=None):
            r = g // 128
            hit = lane1 == g % 128
            if cond is not None:
                hit = hit & cond
            rowv = scr[pl.ds(r, 1), :]
            scr[pl.ds(r, 1), :] = jnp.where(hit, NEG, rowv)
            refresh_chunk(scr, p, r // RB)

        def score_at(g):
            rowv = scrs[0][pl.ds(g // 128, 1), :]
            return jnp.sum(jnp.where(lane1 == g % 128, rowv, 0.0))

        tops = []
        for _ in range(6):
            tops.append(pick(scrs[0], ps[0]))
        for g, m in tops:
            rowv = scrs[0][pl.ds(g // 128, 1), :]
            scrs[0][pl.ds(g // 128, 1), :] = jnp.where(lane1 == g % 128, m,
                                                       rowv)

        loss = jnp.float32(0.0)
        for t in range(T):
            tt = tgt_sm[t]
            s_t = score_at(tt)

            in5 = tops[0][0] == tt
            for i in range(1, 5):
                in5 = in5 | (tops[i][0] == tt)
            contrib = jnp.float32(0.0)
            for i in range(5):
                contrib += jnp.where(tops[i][0] == tt, 0.0,
                                     jax.nn.sigmoid(tops[i][1] - s_t))
            contrib += jnp.where(in5, jax.nn.sigmoid(tops[5][1] - s_t), 0.0)

            scr, p = scrs[1 + t], ps[1 + t]
            exclude(scr, p, tt)
            for i in range(5):
                exclude(scr, p, tops[i][0])
            exclude(scr, p, tops[5][0], cond=in5)
            for _ in range(5):
                g, _m = pick(scr, p)
                contrib += jax.nn.sigmoid(score_at(g) - s_t)

            loss += contrib
        out_ref[...] = jnp.broadcast_to(loss, (1, 1))


def kernel(items_emb, user_emb, target_items):
    xt = jnp.transpose(items_emb)
    grid_spec = pltpu.PrefetchScalarGridSpec(
        num_scalar_prefetch=1,
        grid=(NB,),
        in_specs=[
            pl.BlockSpec((D, B), lambda k, tgt: (0, k)),
            pl.BlockSpec((1, D), lambda k, tgt: (0, 0)),
            pl.BlockSpec(memory_space=pltpu.MemorySpace.HBM),
        ],
        out_specs=pl.BlockSpec((1, 1), lambda k, tgt: (0, 0)),
        scratch_shapes=(
            [pltpu.VMEM((R, 128), jnp.float32) for _ in range(NC)]
            + [pltpu.VMEM((CH, 128), jnp.float32) for _ in range(NC)]
            + [pltpu.VMEM((D, 16), jnp.float32),
               pltpu.VMEM((T, D, 128), jnp.float32),
               pltpu.SemaphoreType.DMA]
        ),
    )
    out = pl.pallas_call(
        _body,
        grid_spec=grid_spec,
        out_shape=jax.ShapeDtypeStruct((1, 1), jnp.float32),
    )(target_items, xt, user_emb, xt)
    return out[0, 0]
